# Initial kernel scaffold; baseline (speedup 1.0000x reference)
import jax
import jax.numpy as jnp
from jax.experimental import pallas as pl


def kernel(x, u, batch, edge_index, W_node, b_node):
    raise NotImplementedError("write your pallas kernel here")


# speedup vs baseline: 38.1675x; 38.1675x over previous
import functools

import jax
import jax.numpy as jnp
from jax import lax
from jax.experimental import pallas as pl
from jax.experimental.pallas import tpu as pltpu

_TILE = 512
_NEG_INF = float("-inf")


def _gelu_exact(v):
    return v * 0.5 * (1.0 + lax.erf(v * 0.7071067811865476))


def _stage1(x_ref, w_ref, b_ref, inst_ref, batch_ref, p_ref, mx_ref, d_ref,
            macc, sacc, cacc, *, n_nodes, n_graphs, m_inst, tile):
    t = pl.program_id(0)
    ntiles = pl.num_programs(0)

    @pl.when(t == 0)
    def _init():
        macc[...] = jnp.full_like(macc[...], _NEG_INF)
        sacc[...] = jnp.zeros_like(sacc[...])
        cacc[...] = jnp.zeros_like(cacc[...])

    bt = batch_ref[0]
    lane = lax.broadcasted_iota(jnp.int32, (1, tile), 1)
    valid = (t * tile + lane) < n_nodes
    gl = lax.broadcasted_iota(jnp.int32, (1, n_graphs), 1)

    xl = x_ref[...]
    h = lax.dot_general(xl, w_ref[...], (((1,), (1,)), ((), ())),
                        preferred_element_type=jnp.float32) + b_ref[...]
    h = _gelu_exact(h)

    g0 = jnp.min(bt)
    g1 = jnp.max(bt)

    def body(g, p_acc):
        inst_g = inst_ref[pl.ds(g * m_inst, m_inst), :]
        pt = lax.dot_general(inst_g, h, (((1,), (1,)), ((), ())),
                             preferred_element_type=jnp.float32)
        mask = (bt == g) & valid
        oh = gl == g

        mg = jnp.max(jnp.where(oh, macc[...], _NEG_INF), axis=1, keepdims=True)
        sg = jnp.sum(jnp.where(oh, sacc[...], 0.0), axis=1, keepdims=True)
        masked_pt = jnp.where(mask, pt, _NEG_INF)
        m_t = jnp.max(masked_pt, axis=1, keepdims=True)
        new_m = jnp.maximum(mg, m_t)
        safe = jnp.where(new_m == _NEG_INF, 0.0, new_m)
        e_old = jnp.where(mg == _NEG_INF, 0.0, jnp.exp(mg - safe))
        s_t = jnp.sum(jnp.where(mask, jnp.exp(pt - safe), 0.0),
                      axis=1, keepdims=True)
        new_s = sg * e_old + s_t
        cnt_t = jnp.sum(jnp.where(mask, 1.0, 0.0))

        macc[...] = jnp.where(oh, new_m, macc[...])
        sacc[...] = jnp.where(oh, new_s, sacc[...])
        cacc[...] = jnp.where(oh, cacc[...] + cnt_t, cacc[...])
        return jnp.where(mask, pt, p_acc)

    p_ref[...] = lax.fori_loop(g0, g1 + 1, body,
                               jnp.zeros((m_inst, tile), jnp.float32))

    @pl.when(t == ntiles - 1)
    def _finalize():
        m_ = macc[...]
        s_ = sacc[...]
        c_ = cacc[...]
        cmax = jnp.max(c_)
        has_pad = c_ < cmax
        mx = jnp.where(has_pad, jnp.maximum(m_, 0.0), m_)
        pad = jnp.where(has_pad, (cmax - c_) * jnp.exp(-mx), 0.0)
        mx_ref[...] = mx
        d_ref[...] = s_ * jnp.exp(m_ - mx) + pad


def _stage2(p_ref, batch_ref, mx_ref, d_ref, out_ref, *, n_nodes, n_graphs,
            tile):
    t = pl.program_id(0)
    bt = batch_ref[0]
    lane = lax.broadcasted_iota(jnp.int32, (1, tile), 1)
    valid = (t * tile + lane) < n_nodes
    gl = lax.broadcasted_iota(jnp.int32, (1, n_graphs), 1)
    pt = p_ref[...]

    g0 = jnp.min(bt)
    g1 = jnp.max(bt)

    def body(g, acc):
        oh = gl == g
        mxg = jnp.max(jnp.where(oh, mx_ref[...], _NEG_INF), axis=1,
                      keepdims=True)
        dg = jnp.sum(jnp.where(oh, d_ref[...], 0.0), axis=1, keepdims=True)
        mask = (bt == g) & valid
        e = jnp.exp(pt - mxg) / dg
        colsum = jnp.sum(e, axis=0, keepdims=True)
        return jnp.where(mask, colsum, acc)

    out_ref[0] = lax.fori_loop(g0, g1 + 1, body,
                               jnp.zeros((1, tile), jnp.float32))


def kernel(x, u, batch, edge_index, W_node, b_node):
    n_nodes, dn = x.shape
    m_inst, n_graphs, dq = u.shape
    tile = _TILE
    ntiles = -(-n_nodes // tile)
    npad = ntiles * tile

    xp = jnp.pad(x, ((0, npad - n_nodes), (0, 0)))
    bp = jnp.pad(batch.astype(jnp.int32), (0, npad - n_nodes),
                 mode="edge").reshape(ntiles, 1, tile)
    inst = jnp.transpose(u, (1, 0, 2)).reshape(n_graphs * m_inst, dq)
    b2 = b_node.reshape(1, dq).astype(jnp.float32)

    s1 = functools.partial(_stage1, n_nodes=n_nodes, n_graphs=n_graphs,
                           m_inst=m_inst, tile=tile)
    p_t, mx, dd = pl.pallas_call(
        s1,
        grid=(ntiles,),
        in_specs=[
            pl.BlockSpec((tile, dn), lambda i: (i, 0)),
            pl.BlockSpec((dq, dn), lambda i: (0, 0)),
            pl.BlockSpec((1, dq), lambda i: (0, 0)),
            pl.BlockSpec((n_graphs * m_inst, dq), lambda i: (0, 0)),
            pl.BlockSpec((1, 1, tile), lambda i: (i, 0, 0)),
        ],
        out_specs=[
            pl.BlockSpec((m_inst, tile), lambda i: (0, i)),
            pl.BlockSpec((m_inst, n_graphs), lambda i: (0, 0)),
            pl.BlockSpec((m_inst, n_graphs), lambda i: (0, 0)),
        ],
        out_shape=[
            jax.ShapeDtypeStruct((m_inst, npad), jnp.float32),
            jax.ShapeDtypeStruct((m_inst, n_graphs), jnp.float32),
            jax.ShapeDtypeStruct((m_inst, n_graphs), jnp.float32),
        ],
        scratch_shapes=[
            pltpu.VMEM((m_inst, n_graphs), jnp.float32),
            pltpu.VMEM((m_inst, n_graphs), jnp.float32),
            pltpu.VMEM((m_inst, n_graphs), jnp.float32),
        ],
    )(xp, W_node, b2, inst, bp)

    s2 = functools.partial(_stage2, n_nodes=n_nodes, n_graphs=n_graphs,
                           tile=tile)
    gate3 = pl.pallas_call(
        s2,
        grid=(ntiles,),
        in_specs=[
            pl.BlockSpec((m_inst, tile), lambda i: (0, i)),
            pl.BlockSpec((1, 1, tile), lambda i: (i, 0, 0)),
            pl.BlockSpec((m_inst, n_graphs), lambda i: (0, 0)),
            pl.BlockSpec((m_inst, n_graphs), lambda i: (0, 0)),
        ],
        out_specs=pl.BlockSpec((1, 1, tile), lambda i: (i, 0, 0)),
        out_shape=jax.ShapeDtypeStruct((ntiles, 1, tile), jnp.float32),
    )(p_t, bp, mx, dd)

    gate = gate3.reshape(npad)[:n_nodes]
    return (jax.nn.sigmoid(gate) > 0.5).astype(x.dtype)
